# baseline (speedup 1.0000x reference)
import jax
import jax.numpy as jnp
from jax.experimental import pallas as pl

BATCH = 16384
DIM = 256
HTOT = 768
BLK = 1024


def _fused_kernel(lab_ref, x_ref, w1t_ref, b1_ref, w2_ref, b2_ref, out_ref):
    x = x_ref[...]
    h = jnp.dot(x, w1t_ref[...], preferred_element_type=jnp.float32)
    h = jnp.maximum(h + b1_ref[...], 0.0)
    y4 = jnp.dot(h, w2_ref[...], preferred_element_type=jnp.float32)
    lab = lab_ref[0, 0, :]
    onehot = (lab[:, None] == jax.lax.broadcasted_iota(jnp.int32, (1, 4), 1)
              ).astype(jnp.float32)
    y = jnp.sum((y4 + b2_ref[...]) * onehot, axis=1, keepdims=True)
    out_ref[...] = y


def kernel(x, group_labels,
           W1_sc, b1_sc, W2_sc, b2_sc,
           W1_st, b1_st, W2_st, b2_st,
           W1_w, b1_w, W2_w, b2_w,
           W1_c, b1_c, W2_c, b2_c):
    w1t = jnp.concatenate([W1_sc, W1_st, W1_w, W1_c], axis=0).T
    b1 = jnp.concatenate([b1_sc, b1_st, b1_w, b1_c])[None, :]
    w2 = jnp.zeros((HTOT, 4), jnp.float32)
    w2 = w2.at[0:128, 0].set(W2_sc[0])
    w2 = w2.at[128:256, 1].set(W2_st[0])
    w2 = w2.at[256:512, 2].set(W2_w[0])
    w2 = w2.at[512:768, 3].set(W2_c[0])
    b2 = jnp.stack([b2_sc[0], b2_st[0], b2_w[0], b2_c[0]])[None, :]
    nblk = BATCH // BLK
    labs = group_labels.astype(jnp.int32).reshape(nblk, 1, BLK)

    out = pl.pallas_call(
        _fused_kernel,
        grid=(nblk,),
        in_specs=[
            pl.BlockSpec((1, 1, BLK), lambda i: (i, 0, 0)),
            pl.BlockSpec((BLK, DIM), lambda i: (i, 0)),
            pl.BlockSpec((DIM, HTOT), lambda i: (0, 0)),
            pl.BlockSpec((1, HTOT), lambda i: (0, 0)),
            pl.BlockSpec((HTOT, 4), lambda i: (0, 0)),
            pl.BlockSpec((1, 4), lambda i: (0, 0)),
        ],
        out_specs=pl.BlockSpec((BLK, 1), lambda i: (i, 0)),
        out_shape=jax.ShapeDtypeStruct((BATCH, 1), jnp.float32),
    )(labs, x, w1t, b1, w2, b2)
    return out


# speedup vs baseline: 1.0453x; 1.0453x over previous
import jax
import jax.numpy as jnp
from jax.experimental import pallas as pl

BATCH = 16384
DIM = 256
HTOT = 768
BLK = 1024


def _fused_kernel(lab_ref, x_ref, w1t_ref, b1_ref, w2_ref, b2_ref, out_ref):
    x = x_ref[...].astype(jnp.bfloat16)
    h = jnp.dot(x, w1t_ref[...], preferred_element_type=jnp.float32)
    h = jnp.maximum(h + b1_ref[...], 0.0)
    y4 = jnp.dot(h.astype(jnp.bfloat16), w2_ref[...],
                 preferred_element_type=jnp.float32)
    lab = lab_ref[0, 0, :]
    onehot = (lab[:, None] == jax.lax.broadcasted_iota(jnp.int32, (1, 4), 1)
              ).astype(jnp.float32)
    y = jnp.sum((y4 + b2_ref[...]) * onehot, axis=1, keepdims=True)
    out_ref[...] = y


def kernel(x, group_labels,
           W1_sc, b1_sc, W2_sc, b2_sc,
           W1_st, b1_st, W2_st, b2_st,
           W1_w, b1_w, W2_w, b2_w,
           W1_c, b1_c, W2_c, b2_c):
    w1t = jnp.concatenate([W1_sc, W1_st, W1_w, W1_c], axis=0).T.astype(jnp.bfloat16)
    b1 = jnp.concatenate([b1_sc, b1_st, b1_w, b1_c])[None, :]
    w2 = jnp.zeros((HTOT, 4), jnp.float32)
    w2 = w2.at[0:128, 0].set(W2_sc[0])
    w2 = w2.at[128:256, 1].set(W2_st[0])
    w2 = w2.at[256:512, 2].set(W2_w[0])
    w2 = w2.at[512:768, 3].set(W2_c[0])
    w2 = w2.astype(jnp.bfloat16)
    b2 = jnp.stack([b2_sc[0], b2_st[0], b2_w[0], b2_c[0]])[None, :]
    nblk = BATCH // BLK
    labs = group_labels.astype(jnp.int32).reshape(nblk, 1, BLK)

    out = pl.pallas_call(
        _fused_kernel,
        grid=(nblk,),
        in_specs=[
            pl.BlockSpec((1, 1, BLK), lambda i: (i, 0, 0)),
            pl.BlockSpec((BLK, DIM), lambda i: (i, 0)),
            pl.BlockSpec((DIM, HTOT), lambda i: (0, 0)),
            pl.BlockSpec((1, HTOT), lambda i: (0, 0)),
            pl.BlockSpec((HTOT, 4), lambda i: (0, 0)),
            pl.BlockSpec((1, 4), lambda i: (0, 0)),
        ],
        out_specs=pl.BlockSpec((BLK, 1), lambda i: (i, 0)),
        out_shape=jax.ShapeDtypeStruct((BATCH, 1), jnp.float32),
    )(labs, x, w1t, b1, w2, b2)
    return out


# speedup vs baseline: 1.2013x; 1.1492x over previous
import jax
import jax.numpy as jnp
from jax.experimental import pallas as pl

BATCH = 16384
DIM = 256
HTOT = 768
BLK = 1024


def _fused_kernel(lab_ref, x_ref, w1t_ref, b1_ref, w2_ref, b2_ref, out_ref):
    x = x_ref[...].astype(jnp.bfloat16)
    h = jnp.dot(x, w1t_ref[...], preferred_element_type=jnp.float32)
    h = jnp.maximum(h + b1_ref[...], 0.0)
    lab = lab_ref[0, 0, :]
    j = jax.lax.broadcasted_iota(jnp.int32, (1, HTOT), 1)
    seg = ((j >= 128).astype(jnp.int32) + (j >= 256).astype(jnp.int32)
           + (j >= 512).astype(jnp.int32))
    wsel = jnp.where(lab[:, None] == seg, w2_ref[...], 0.0)
    y = jnp.sum(h * wsel, axis=1, keepdims=True)
    onehot = (lab[:, None] == jax.lax.broadcasted_iota(jnp.int32, (1, 4), 1)
              ).astype(jnp.float32)
    b2sel = jnp.sum(onehot * b2_ref[...], axis=1, keepdims=True)
    out_ref[...] = y + b2sel


def kernel(x, group_labels,
           W1_sc, b1_sc, W2_sc, b2_sc,
           W1_st, b1_st, W2_st, b2_st,
           W1_w, b1_w, W2_w, b2_w,
           W1_c, b1_c, W2_c, b2_c):
    w1t = jnp.concatenate([W1_sc, W1_st, W1_w, W1_c], axis=0).T.astype(jnp.bfloat16)
    b1 = jnp.concatenate([b1_sc, b1_st, b1_w, b1_c])[None, :]
    w2all = jnp.concatenate([W2_sc[0], W2_st[0], W2_w[0], W2_c[0]])[None, :]
    b2 = jnp.stack([b2_sc[0], b2_st[0], b2_w[0], b2_c[0]])[None, :]
    nblk = BATCH // BLK
    labs = group_labels.astype(jnp.int32).reshape(nblk, 1, BLK)

    out = pl.pallas_call(
        _fused_kernel,
        grid=(nblk,),
        in_specs=[
            pl.BlockSpec((1, 1, BLK), lambda i: (i, 0, 0)),
            pl.BlockSpec((BLK, DIM), lambda i: (i, 0)),
            pl.BlockSpec((DIM, HTOT), lambda i: (0, 0)),
            pl.BlockSpec((1, HTOT), lambda i: (0, 0)),
            pl.BlockSpec((1, HTOT), lambda i: (0, 0)),
            pl.BlockSpec((1, 4), lambda i: (0, 0)),
        ],
        out_specs=pl.BlockSpec((BLK, 1), lambda i: (i, 0)),
        out_shape=jax.ShapeDtypeStruct((BATCH, 1), jnp.float32),
    )(labs, x, w1t, b1, w2all, b2)
    return out


# speedup vs baseline: 1.4335x; 1.1933x over previous
import jax
import jax.numpy as jnp
from jax import lax
from jax.experimental import pallas as pl
from jax.experimental.pallas import tpu as pltpu

BATCH = 16384
DIM = 256
BLK = 1024

_NT = (((1,), (1,)), ((), ()))


def _rowsum(p):
    return jnp.sum(p, axis=1, keepdims=True)


def _fused_kernel(lab_ref, x_ref,
                  w1sc_ref, b1sc_ref, w2sc_ref, b2sc_ref,
                  w1st_ref, b1st_ref, w2st_ref, b2st_ref,
                  w1w_ref, b1w_ref, w2w_ref, b2w_ref,
                  w1c_ref, b1c_ref, w2c_ref, b2c_ref,
                  out_ref):
    xb = x_ref[...].astype(jnp.bfloat16)

    w1a = jnp.concatenate([w1sc_ref[...], w1st_ref[...]], axis=0)
    b1a = jnp.concatenate([b1sc_ref[...], b1st_ref[...]])
    w2a = jnp.concatenate([w2sc_ref[...], w2st_ref[...]], axis=1)
    ha = lax.dot_general(xb, w1a.astype(jnp.bfloat16), _NT,
                         preferred_element_type=jnp.float32)
    pa = jnp.maximum(ha + b1a[None, :], 0.0) * w2a
    s_sc = _rowsum(pa[:, :128])
    s_st = _rowsum(pa[:, 128:])

    hw = lax.dot_general(xb, w1w_ref[...].astype(jnp.bfloat16), _NT,
                         preferred_element_type=jnp.float32)
    s_w = _rowsum(jnp.maximum(hw + b1w_ref[...][None, :], 0.0) * w2w_ref[...])

    hc = lax.dot_general(xb, w1c_ref[...].astype(jnp.bfloat16), _NT,
                         preferred_element_type=jnp.float32)
    s_c = _rowsum(jnp.maximum(hc + b1c_ref[...][None, :], 0.0) * w2c_ref[...])

    lab = lab_ref[...][:, None]
    y = ((lab == 0) * (s_sc + b2sc_ref[0])
         + (lab == 1) * (s_st + b2st_ref[0])
         + (lab == 2) * (s_w + b2w_ref[0])
         + (lab == 3) * (s_c + b2c_ref[0]))
    out_ref[...] = y


def kernel(x, group_labels,
           W1_sc, b1_sc, W2_sc, b2_sc,
           W1_st, b1_st, W2_st, b2_st,
           W1_w, b1_w, W2_w, b2_w,
           W1_c, b1_c, W2_c, b2_c):
    nblk = BATCH // BLK

    def full(a):
        return pl.BlockSpec(a.shape, lambda i: (0,) * a.ndim)

    def smem1(a):
        return pl.BlockSpec(memory_space=pltpu.SMEM)

    out = pl.pallas_call(
        _fused_kernel,
        grid=(nblk,),
        in_specs=[
            pl.BlockSpec((BLK,), lambda i: (i,)),
            pl.BlockSpec((BLK, DIM), lambda i: (i, 0)),
            full(W1_sc), full(b1_sc), full(W2_sc), smem1(b2_sc),
            full(W1_st), full(b1_st), full(W2_st), smem1(b2_st),
            full(W1_w), full(b1_w), full(W2_w), smem1(b2_w),
            full(W1_c), full(b1_c), full(W2_c), smem1(b2_c),
        ],
        out_specs=pl.BlockSpec((BLK, 1), lambda i: (i, 0)),
        out_shape=jax.ShapeDtypeStruct((BATCH, 1), jnp.float32),
    )(group_labels, x,
      W1_sc, b1_sc, W2_sc, b2_sc,
      W1_st, b1_st, W2_st, b2_st,
      W1_w, b1_w, W2_w, b2_w,
      W1_c, b1_c, W2_c, b2_c)
    return out


# speedup vs baseline: 1.5276x; 1.0656x over previous
import jax
import jax.numpy as jnp
from jax import lax
from jax.experimental import pallas as pl
from jax.experimental.pallas import tpu as pltpu

BATCH = 16384
DIM = 256
BLK = 1024

_NT = (((1,), (1,)), ((), ()))


def _fused_kernel(lab_ref, x_ref,
                  w1sc_ref, b1sc_ref, w2sc_ref, b2sc_ref,
                  w1st_ref, b1st_ref, w2st_ref, b2st_ref,
                  w1w_ref, b1w_ref, w2w_ref, b2w_ref,
                  w1c_ref, b1c_ref, w2c_ref, b2c_ref,
                  out_ref):
    xb = x_ref[...].astype(jnp.bfloat16)

    w1a = jnp.concatenate([w1sc_ref[...], w1st_ref[...]], axis=0)
    b1a = jnp.concatenate([b1sc_ref[...], b1st_ref[...]])
    w2a = jnp.concatenate([w2sc_ref[...], w2st_ref[...]], axis=1)

    ha = lax.dot_general(w1a.astype(jnp.bfloat16), xb, _NT,
                         preferred_element_type=jnp.float32)
    pa = jnp.maximum(ha + b1a[:, None], 0.0) * w2a[0, :][:, None]
    s_sc = jnp.sum(pa[:128, :], axis=0, keepdims=True)
    s_st = jnp.sum(pa[128:, :], axis=0, keepdims=True)

    hw = lax.dot_general(w1w_ref[...].astype(jnp.bfloat16), xb, _NT,
                         preferred_element_type=jnp.float32)
    pw = jnp.maximum(hw + b1w_ref[...][:, None], 0.0) * w2w_ref[0, :][:, None]
    s_w = jnp.sum(pw, axis=0, keepdims=True)

    hc = lax.dot_general(w1c_ref[...].astype(jnp.bfloat16), xb, _NT,
                         preferred_element_type=jnp.float32)
    pc = jnp.maximum(hc + b1c_ref[...][:, None], 0.0) * w2c_ref[0, :][:, None]
    s_c = jnp.sum(pc, axis=0, keepdims=True)

    lab = lab_ref[...][None, :]
    y = ((lab == 0) * (s_sc + b2sc_ref[0])
         + (lab == 1) * (s_st + b2st_ref[0])
         + (lab == 2) * (s_w + b2w_ref[0])
         + (lab == 3) * (s_c + b2c_ref[0]))
    out_ref[...] = y.reshape(BLK, 1)


def kernel(x, group_labels,
           W1_sc, b1_sc, W2_sc, b2_sc,
           W1_st, b1_st, W2_st, b2_st,
           W1_w, b1_w, W2_w, b2_w,
           W1_c, b1_c, W2_c, b2_c):
    nblk = BATCH // BLK

    def full(a):
        return pl.BlockSpec(a.shape, lambda i: (0,) * a.ndim)

    def smem1(a):
        return pl.BlockSpec(memory_space=pltpu.SMEM)

    out = pl.pallas_call(
        _fused_kernel,
        grid=(nblk,),
        in_specs=[
            pl.BlockSpec((BLK,), lambda i: (i,)),
            pl.BlockSpec((BLK, DIM), lambda i: (i, 0)),
            full(W1_sc), full(b1_sc), full(W2_sc), smem1(b2_sc),
            full(W1_st), full(b1_st), full(W2_st), smem1(b2_st),
            full(W1_w), full(b1_w), full(W2_w), smem1(b2_w),
            full(W1_c), full(b1_c), full(W2_c), smem1(b2_c),
        ],
        out_specs=pl.BlockSpec((BLK, 1), lambda i: (i, 0)),
        out_shape=jax.ShapeDtypeStruct((BATCH, 1), jnp.float32),
    )(group_labels, x,
      W1_sc, b1_sc, W2_sc, b2_sc,
      W1_st, b1_st, W2_st, b2_st,
      W1_w, b1_w, W2_w, b2_w,
      W1_c, b1_c, W2_c, b2_c)
    return out
